# Initial kernel scaffold; baseline (speedup 1.0000x reference)
import jax
import jax.numpy as jnp
from jax.experimental import pallas as pl


def kernel(x, W_patch, b_patch, ln1_g, ln1_b, Wq, Wk, Wv, Wo, bo, Wp, bp, ln2_g, ln2_b, ln3_g, ln3_b, Wr1, br1, Wn1, bn1, W1a, b1a, W1b, b1b, Wr2, br2, Wn2, bn2, W2a, b2a, W2b, b2b, Wc, bc):
    raise NotImplementedError("write your pallas kernel here")


# speedup vs baseline: 1.1834x; 1.1834x over previous
import functools

import numpy as np
import jax
import jax.numpy as jnp
from jax.experimental import pallas as pl
from jax.experimental.pallas import tpu as pltpu

IMG = 224
P = 14
NP = 256
PD = 196
D = 512
H = 8
HS = 64
OD = 1024
E = 8
TOPK = 2
FF = 4096
B = 4
T = B * NP
TILE = 128
FCH = 1024
NJ = FF // FCH
NEG = -jnp.inf
PH = jax.lax.Precision.HIGHEST


def _pos_enc(seq, d):
    pos = np.arange(seq)[:, None].astype(np.float64)
    i = np.arange(d)[None, :]
    ang = pos / np.power(10000.0, (2 * (i // 2)) / d)
    pe = np.zeros((seq, d), np.float32)
    pe[:, 0::2] = np.sin(ang[:, 0::2])
    pe[:, 1::2] = np.cos(ang[:, 1::2])
    return pe


def _ln(x, g, b):
    m = jnp.mean(x, -1, keepdims=True)
    v = jnp.mean((x - m) ** 2, -1, keepdims=True)
    return (x - m) / jnp.sqrt(v + 1e-5) * g + b


def _attn_body(p_ref, wp_ref, bp_ref, pe_ref, g1_ref, b1_ref, wq_ref, wk_ref,
               wv_ref, wo_ref, bo_ref, h1_ref):
    x = p_ref[0]
    h0 = jnp.dot(x, wp_ref[...], preferred_element_type=jnp.float32)
    h0 = h0 + bp_ref[...] + pe_ref[...]
    xa = _ln(h0, g1_ref[...], b1_ref[...])
    q = jnp.dot(xa, wq_ref[...], preferred_element_type=jnp.float32)
    k = jnp.dot(xa, wk_ref[...], preferred_element_type=jnp.float32)
    v = jnp.dot(xa, wv_ref[...], preferred_element_type=jnp.float32)
    row = jax.lax.broadcasted_iota(jnp.int32, (NP, NP), 0)
    col = jax.lax.broadcasted_iota(jnp.int32, (NP, NP), 1)
    causal = row >= col
    outs = []
    for h in range(H):
        qh = q[:, h * HS:(h + 1) * HS]
        kh = k[:, h * HS:(h + 1) * HS]
        vh = v[:, h * HS:(h + 1) * HS]
        s = jax.lax.dot_general(qh, kh, (((1,), (1,)), ((), ())),
                                preferred_element_type=jnp.float32)
        s = s * (HS ** -0.5)
        s = jnp.where(causal, s, NEG)
        m = jnp.max(s, axis=1, keepdims=True)
        e = jnp.exp(s - m)
        a = e / jnp.sum(e, axis=1, keepdims=True)
        outs.append(jnp.dot(a, vh, preferred_element_type=jnp.float32))
    o = jnp.concatenate(outs, axis=1)
    h1_ref[0] = h0 + jnp.dot(o, wo_ref[...],
                             preferred_element_type=jnp.float32) + bo_ref[...]


def _run_attn(p3, W_patch, b_patch, pe, ln1_g, ln1_b, Wq, Wk, Wv, Wo, bo):
    full2 = lambda a: pl.BlockSpec(a.shape, lambda b: (0,) * a.ndim)
    return pl.pallas_call(
        _attn_body,
        grid=(B,),
        in_specs=[
            pl.BlockSpec((1, NP, PD), lambda b: (b, 0, 0)),
            full2(W_patch), full2(b_patch), full2(pe), full2(ln1_g),
            full2(ln1_b), full2(Wq), full2(Wk), full2(Wv), full2(Wo),
            full2(bo),
        ],
        out_specs=pl.BlockSpec((1, NP, D), lambda b: (b, 0, 0)),
        out_shape=jax.ShapeDtypeStruct((B, NP, D), jnp.float32),
    )(p3, W_patch, b_patch, pe, ln1_g, ln1_b, Wq, Wk, Wv, Wo, bo)


def _proj_body(h1_ref, wp_ref, bp_ref, g2_ref, b2_ref, g3_ref, b3_ref,
               wr1_ref, br1_ref, wn1_ref, bn1_ref, nz1_ref,
               wr2_ref, br2_ref, wn2_ref, bn2_ref, nz2_ref,
               xfb1_ref, xfb2_ref, noisy1_ref, noisy2_ref):
    h2 = jnp.dot(h1_ref[...], wp_ref[...],
                 preferred_element_type=jnp.float32) + bp_ref[...]

    def router(g, b, wr, br, wn, bn, nz):
        xf = _ln(h2, g, b)
        logits = jnp.dot(xf, wr, preferred_element_type=jnp.float32) + br
        raw = jnp.dot(xf, wn, preferred_element_type=jnp.float32) + bn
        sp = jnp.maximum(raw, 0.0) + jnp.log1p(jnp.exp(-jnp.abs(raw)))
        return xf, logits + nz * sp

    xf1, n1 = router(g2_ref[...], b2_ref[...], wr1_ref[...], br1_ref[...],
                     wn1_ref[...], bn1_ref[...], nz1_ref[...])
    xf2, n2 = router(g3_ref[...], b3_ref[...], wr2_ref[...], br2_ref[...],
                     wn2_ref[...], bn2_ref[...], nz2_ref[...])
    xfb1_ref[...] = xf1.astype(jnp.bfloat16)
    xfb2_ref[...] = xf2.astype(jnp.bfloat16)
    noisy1_ref[...] = n1
    noisy2_ref[...] = n2


def _run_proj(h1f, Wp, bp, ln2_g, ln2_b, ln3_g, ln3_b,
              Wr1, br1, Wn1, bn1, nz1, Wr2, br2, Wn2, bn2, nz2):
    args = (h1f, Wp, bp, ln2_g, ln2_b, ln3_g, ln3_b,
            Wr1, br1, Wn1, bn1, nz1, Wr2, br2, Wn2, bn2, nz2)
    return pl.pallas_call(
        _proj_body,
        in_specs=[pl.BlockSpec(a.shape, lambda: (0,) * a.ndim) for a in args],
        out_specs=[
            pl.BlockSpec((T, OD), lambda: (0, 0)),
            pl.BlockSpec((T, OD), lambda: (0, 0)),
            pl.BlockSpec((T, E), lambda: (0, 0)),
            pl.BlockSpec((T, E), lambda: (0, 0)),
        ],
        out_shape=[
            jax.ShapeDtypeStruct((T, OD), jnp.bfloat16),
            jax.ShapeDtypeStruct((T, OD), jnp.bfloat16),
            jax.ShapeDtypeStruct((T, E), jnp.float32),
            jax.ShapeDtypeStruct((T, E), jnp.float32),
        ],
    )(*args)


def _route_body(noisy_ref, rankm_ref, gate_ref, counts_ref):
    noisy = noisy_ref[...]
    eio = jax.lax.broadcasted_iota(jnp.int32, (T, E), 1)
    m1 = jnp.max(noisy, axis=1, keepdims=True)
    i1 = jnp.min(jnp.where(noisy == m1, eio, E), axis=1, keepdims=True)
    n2 = jnp.where(eio == i1, NEG, noisy)
    m2 = jnp.max(n2, axis=1, keepdims=True)
    i2 = jnp.min(jnp.where(n2 == m2, eio, E), axis=1, keepdims=True)
    maskb = (eio == i1) | (eio == i2)
    ez = jnp.where(maskb, jnp.exp(noisy - m1), 0.0)
    gate_ref[...] = ez / jnp.sum(ez, axis=1, keepdims=True)
    maskf = maskb.astype(jnp.float32)
    r = jax.lax.broadcasted_iota(jnp.int32, (T, T), 0)
    c = jax.lax.broadcasted_iota(jnp.int32, (T, T), 1)
    tri = (c < r).astype(jnp.float32)
    rank = jnp.dot(tri, maskf, preferred_element_type=jnp.float32)
    rankm_ref[...] = jnp.where(maskb, rank, -1.0)
    counts_ref[...] = jnp.sum(maskf, axis=0, keepdims=True).astype(jnp.int32)


def _run_route(noisy):
    return pl.pallas_call(
        _route_body,
        in_specs=[pl.BlockSpec((T, E), lambda: (0, 0))],
        out_specs=[
            pl.BlockSpec((T, E), lambda: (0, 0)),
            pl.BlockSpec((T, E), lambda: (0, 0)),
            pl.BlockSpec((1, E), lambda: (0, 0)),
        ],
        out_shape=[
            jax.ShapeDtypeStruct((T, E), jnp.float32),
            jax.ShapeDtypeStruct((T, E), jnp.float32),
            jax.ShapeDtypeStruct((1, E), jnp.int32),
        ],
    )(noisy)


def _moe_body(counts_ref, xfb_ref, rankm_ref, gate_ref, w1_ref, b1_ref,
              w2_ref, b2_ref, out_ref, xs_ref, y_ref):
    e = pl.program_id(0)
    j = pl.program_id(1)

    @pl.when((e == 0) & (j == 0))
    def _init():
        out_ref[...] = jnp.zeros_like(out_ref)

    cnt = counts_ref[e]
    nt = (cnt + (TILE - 1)) // TILE

    @pl.when(cnt > 0)
    def _work():
        oh = (jax.lax.broadcasted_iota(jnp.int32, (E, 1), 0) == e
              ).astype(jnp.float32)
        rank_col = jnp.dot(rankm_ref[...], oh, precision=PH,
                           preferred_element_type=jnp.float32)
        sio = jax.lax.broadcasted_iota(jnp.int32, (1, TILE), 1
                                       ).astype(jnp.float32)

        @pl.when(j == 0)
        def _gather():
            def body(rt, _):
                svec = sio + (rt * TILE).astype(jnp.float32)
                dt = (rank_col == svec).astype(jnp.bfloat16)
                xs = jax.lax.dot_general(
                    dt, xfb_ref[...], (((0,), (0,)), ((), ())),
                    preferred_element_type=jnp.float32)
                xs_ref[pl.ds(rt * TILE, TILE), :] = xs.astype(jnp.bfloat16)
                return 0
            jax.lax.fori_loop(0, nt, body, 0)

        w1c = w1_ref[0].astype(jnp.bfloat16)
        w2c = w2_ref[0].astype(jnp.bfloat16)
        b1c = b1_ref[0]

        def mlp(rt, _):
            xs = xs_ref[pl.ds(rt * TILE, TILE), :]
            h = jnp.dot(xs, w1c, preferred_element_type=jnp.float32) + b1c
            h = jnp.maximum(h, 0.0).astype(jnp.bfloat16)
            yc = jnp.dot(h, w2c, preferred_element_type=jnp.float32)
            sl = pl.ds(rt * TILE, TILE)
            if_first = j == 0

            @pl.when(if_first)
            def _():
                y_ref[sl, :] = yc

            @pl.when(jnp.logical_not(if_first))
            def _():
                y_ref[sl, :] = y_ref[sl, :] + yc
            return 0
        jax.lax.fori_loop(0, nt, mlp, 0)

        @pl.when(j == NJ - 1)
        def _combine():
            gate_col = jnp.dot(gate_ref[...], oh, precision=PH,
                               preferred_element_type=jnp.float32)
            b2c = b2_ref[0]

            def body(rt, _):
                sl = pl.ds(rt * TILE, TILE)
                svec = sio + (rt * TILE).astype(jnp.float32)
                dt = (rank_col == svec).astype(jnp.bfloat16)
                yt = (y_ref[sl, :] + b2c).astype(jnp.bfloat16)
                contrib = jnp.dot(dt, yt,
                                  preferred_element_type=jnp.float32)
                out_ref[...] += contrib * gate_col
                return 0
            jax.lax.fori_loop(0, nt, body, 0)


def _run_moe(counts, xfb, rankm, gate, W1, b1, W2, b2):
    grid_spec = pltpu.PrefetchScalarGridSpec(
        num_scalar_prefetch=1,
        grid=(E, NJ),
        in_specs=[
            pl.BlockSpec((T, OD), lambda e, j, c: (0, 0)),
            pl.BlockSpec((T, E), lambda e, j, c: (0, 0)),
            pl.BlockSpec((T, E), lambda e, j, c: (0, 0)),
            pl.BlockSpec((1, OD, FCH), lambda e, j, c: (e, 0, j)),
            pl.BlockSpec((1, 1, FCH), lambda e, j, c: (e, 0, j)),
            pl.BlockSpec((1, FCH, OD), lambda e, j, c: (e, j, 0)),
            pl.BlockSpec((1, 1, OD), lambda e, j, c: (e, 0, 0)),
        ],
        out_specs=pl.BlockSpec((T, OD), lambda e, j, c: (0, 0)),
        scratch_shapes=[
            pltpu.VMEM((T, OD), jnp.bfloat16),
            pltpu.VMEM((T, OD), jnp.float32),
        ],
    )
    return pl.pallas_call(
        _moe_body,
        grid_spec=grid_spec,
        out_shape=jax.ShapeDtypeStruct((T, OD), jnp.float32),
    )(counts, xfb, rankm, gate, W1, b1.reshape(E, 1, -1), W2,
      b2.reshape(E, 1, -1))


def _tail_body(f2_ref, wc_ref, bc_ref, img_ref, cls_ref):
    sel = (jax.lax.broadcasted_iota(jnp.int32, (B, T), 1) // NP ==
           jax.lax.broadcasted_iota(jnp.int32, (B, T), 0)
           ).astype(jnp.float32) * (1.0 / NP)
    img = jnp.dot(sel, f2_ref[...], preferred_element_type=jnp.float32)
    img_ref[...] = img
    cls_ref[...] = jnp.dot(img, wc_ref[...],
                           preferred_element_type=jnp.float32) + bc_ref[...]


def _run_tail(f2f, Wc, bc):
    return pl.pallas_call(
        _tail_body,
        in_specs=[
            pl.BlockSpec((T, OD), lambda: (0, 0)),
            pl.BlockSpec((OD, 1), lambda: (0, 0)),
            pl.BlockSpec((1, 1), lambda: (0, 0)),
        ],
        out_specs=[
            pl.BlockSpec((B, OD), lambda: (0, 0)),
            pl.BlockSpec((B, 1), lambda: (0, 0)),
        ],
        out_shape=[
            jax.ShapeDtypeStruct((B, OD), jnp.float32),
            jax.ShapeDtypeStruct((B, 1), jnp.float32),
        ],
    )(f2f, Wc, bc)


def kernel(x, W_patch, b_patch, ln1_g, ln1_b, Wq, Wk, Wv, Wo, bo, Wp, bp,
           ln2_g, ln2_b, ln3_g, ln3_b, Wr1, br1, Wn1, bn1, W1a, b1a, W1b,
           b1b, Wr2, br2, Wn2, bn2, W2a, b2a, W2b, b2b, Wc, bc):
    b = x.shape[0]
    g = IMG // P
    p = x.reshape(b, 1, g, P, g, P).transpose(0, 1, 2, 4, 3, 5)
    p = p.reshape(b, 1, NP, PD).transpose(0, 2, 1, 3).reshape(b, NP, PD)

    pe = jnp.asarray(_pos_enc(NP, D))
    nz1 = jax.random.normal(jax.random.fold_in(jax.random.key(42), 1),
                            (T, E), jnp.float32)
    nz2 = jax.random.normal(jax.random.fold_in(jax.random.key(42), 2),
                            (T, E), jnp.float32)

    r1 = lambda a: a.reshape(1, -1)

    h1 = _run_attn(p, W_patch, r1(b_patch), pe, r1(ln1_g), r1(ln1_b),
                   Wq, Wk, Wv, Wo, r1(bo))
    h1f = h1.reshape(T, D)

    xfb1, xfb2, noisy1, noisy2 = _run_proj(
        h1f, Wp, r1(bp), r1(ln2_g), r1(ln2_b), r1(ln3_g), r1(ln3_b),
        Wr1, r1(br1), Wn1, r1(bn1), nz1, Wr2, r1(br2), Wn2, r1(bn2), nz2)

    rank1, gate1, counts1 = _run_route(noisy1)
    rank2, gate2, counts2 = _run_route(noisy2)

    f1f = _run_moe(counts1.reshape(E), xfb1, rank1, gate1, W1a, b1a, W1b, b1b)
    f2f = _run_moe(counts2.reshape(E), xfb2, rank2, gate2, W2a, b2a, W2b, b2b)

    img_vec, cls = _run_tail(f2f, Wc, bc.reshape(1, 1))

    return (f1f.reshape(b, NP, OD), f2f.reshape(b, NP, OD), img_vec, cls)
